# baseline (speedup 1.0000x reference)
import functools
import math

import jax
import jax.numpy as jnp
from jax import lax
from jax.experimental import pallas as pl
from jax.experimental.pallas import tpu as pltpu
from jax.experimental.pallas import tpu_sc as plsc

DIM = 64
SCALE = math.sqrt(DIM)
NC = 2
NS = 16
NW = NC * NS
LANES = 16


def _build(B):
    b_per_w = B // NW
    C = 1024
    K = C // 128
    n_chunks = b_per_w // C
    assert b_per_w % C == 0

    mesh = plsc.VectorSubcoreMesh(core_axis_name="c", subcore_axis_name="s")

    @functools.partial(
        pl.kernel,
        mesh=mesh,
        out_type=jax.ShapeDtypeStruct((B, DIM), jnp.float32),
        scratch_types=[
            pltpu.VMEM((K, 128), jnp.int32),
            pltpu.VMEM((C, DIM), jnp.float32),
            pltpu.SemaphoreType.DMA,
        ],
        compiler_params=pltpu.CompilerParams(use_tc_tiling_on_sc=False),
    )
    def k(idx_hbm, table_hbm, out_hbm, idx_v, rows_v, sem):
        wid = lax.axis_index("s") * NC + lax.axis_index("c")
        row_base = wid * b_per_w

        def chunk_body(ci, carry):
            row0 = row_base + ci * C
            pltpu.sync_copy(
                idx_hbm.at[pl.ds(pl.multiple_of(row0 // 128, 8), K)], idx_v)
            copies = [
                pltpu.async_copy(
                    table_hbm.at[idx_v.at[kk]],
                    rows_v.at[pl.ds(kk * 128, 128)],
                    sem,
                )
                for kk in range(K)
            ]
            for cp in copies:
                cp.wait()

            def scale_row(r, c2):
                for j in range(DIM // LANES):
                    sl = pl.ds(j * LANES, LANES)
                    rows_v[r, sl] = rows_v[r, sl] * SCALE
                return c2

            lax.fori_loop(0, C, scale_row, 0)
            pltpu.sync_copy(rows_v, out_hbm.at[pl.ds(row0, C)])
            return carry

        lax.fori_loop(0, n_chunks, chunk_body, 0)

    return k


def kernel(tokens, embedding_weight):
    S0, S1 = tokens.shape
    B = S0 * S1
    idx = tokens.reshape(B // 128, 128).astype(jnp.int32)
    out = _build(B)(idx, embedding_weight)
    return out.reshape(S0, S1, DIM)


# speedup vs baseline: 1.0907x; 1.0907x over previous
import functools
import math

import jax
import jax.numpy as jnp
from jax import lax
from jax.experimental import pallas as pl
from jax.experimental.pallas import tpu as pltpu
from jax.experimental.pallas import tpu_sc as plsc

DIM = 64
SCALE = math.sqrt(DIM)
NC = 2
NS = 16
NW = NC * NS
LANES = 16
C = 512
K = C // 128


def _build(B):
    b_per_w = B // NW
    n_chunks = b_per_w // C
    assert b_per_w % C == 0 and n_chunks % 2 == 0 and n_chunks >= 4

    mesh = plsc.VectorSubcoreMesh(core_axis_name="c", subcore_axis_name="s")

    @functools.partial(
        pl.kernel,
        mesh=mesh,
        out_type=jax.ShapeDtypeStruct((B, DIM), jnp.float32),
        scratch_types=[
            pltpu.VMEM((K, 128), jnp.int32),
            pltpu.VMEM((K, 128), jnp.int32),
            pltpu.VMEM((C, DIM), jnp.float32),
            pltpu.VMEM((C, DIM), jnp.float32),
            pltpu.SemaphoreType.DMA,
            pltpu.SemaphoreType.DMA,
            pltpu.SemaphoreType.DMA,
            pltpu.SemaphoreType.DMA,
        ],
        compiler_params=pltpu.CompilerParams(use_tc_tiling_on_sc=False),
    )
    def k(idx_hbm, table_hbm, out_hbm, idx0, idx1, rows0, rows1,
          g0, g1, o0, o1):
        wid = lax.axis_index("s") * NC + lax.axis_index("c")
        chunk_base = wid * n_chunks
        row_base = wid * b_per_w

        def fire_gather(c, idx_v, rows_v, sem):
            pltpu.sync_copy(idx_hbm.at[chunk_base + c], idx_v)
            for kk in range(K):
                pltpu.async_copy(
                    table_hbm.at[idx_v.at[kk]],
                    rows_v.at[pl.ds(kk * 128, 128)],
                    sem,
                )

        def drain_gather(rows_v, sem):
            pltpu.make_async_copy(table_hbm.at[pl.ds(0, C)], rows_v, sem).wait()

        def scale(rows_v):
            @plsc.parallel_loop(0, C, unroll=8)
            def _(r):
                for j in range(DIM // LANES):
                    sl = pl.ds(j * LANES, LANES)
                    rows_v[r, sl] = rows_v[r, sl] * SCALE

        def fire_out(c, rows_v, sem):
            pltpu.async_copy(rows_v, out_hbm.at[pl.ds(row_base + c * C, C)], sem)

        def drain_out(rows_v, sem):
            pltpu.make_async_copy(rows_v, out_hbm.at[pl.ds(0, C)], sem).wait()

        bufs = ((idx0, rows0, g0, o0), (idx1, rows1, g1, o1))

        def step(c, bi, ni):
            idx_b, rows_b, g_b, o_b = bufs[bi]
            idx_n, rows_n, g_n, o_n = bufs[ni]
            drain_gather(rows_b, g_b)
            drain_out(rows_n, o_n)
            fire_gather(c + 1, idx_n, rows_n, g_n)
            scale(rows_b)
            fire_out(c, rows_b, o_b)

        fire_gather(0, idx0, rows0, g0)
        drain_gather(rows0, g0)
        fire_gather(1, idx1, rows1, g1)
        scale(rows0)
        fire_out(0, rows0, o0)

        def pair(j, carry):
            step(2 * j + 1, 1, 0)
            step(2 * j + 2, 0, 1)
            return carry

        lax.fori_loop(0, (n_chunks - 2) // 2, pair, 0)

        drain_gather(rows1, g1)
        drain_out(rows0, o0)
        scale(rows1)
        fire_out(n_chunks - 1, rows1, o1)
        drain_out(rows1, o1)

    return k


def kernel(tokens, embedding_weight):
    S0, S1 = tokens.shape
    B = S0 * S1
    idx = tokens.reshape(B // C, K, 128).astype(jnp.int32)
    out = _build(B)(idx, embedding_weight)
    return out.reshape(S0, S1, DIM)
